# baseline (speedup 1.0000x reference)
import functools

import jax
import jax.numpy as jnp
from jax.experimental import pallas as pl
from jax.experimental.pallas import tpu as pltpu


def _msg_block_kernel(e_ref, h_ref, wt_ref, wb_ref, bt_ref, bb_ref, o_ref,
                      *, me):
    e = e_ref[0].astype(jnp.bfloat16)
    h = h_ref[0].astype(jnp.bfloat16)
    top = jnp.dot(wt_ref[...], e, preferred_element_type=jnp.float32)
    bot = jnp.dot(wb_ref[...], h, preferred_element_type=jnp.float32)
    top = jnp.maximum(top + bt_ref[...], 0.0)
    bot = jnp.maximum(bot + bb_ref[...], 0.0)
    o_ref[0, :me] = top
    o_ref[0, me:] = bot


def kernel(e_vw, h_w, wk_e, wk_h, bk):
    B, Fe, N = e_vw.shape
    Fn = h_w.shape[1]
    Mout = wk_e.shape[0]
    me = Mout // 2

    w_top = wk_e[:me].astype(jnp.bfloat16)
    w_bot = wk_h[me:].astype(jnp.bfloat16)
    b_top = bk[:me]
    b_bot = bk[me:]

    grid = (B,)
    out_shape = jax.ShapeDtypeStruct((B, Mout, N), jnp.float32)
    in_specs = [
        pl.BlockSpec((1, Fe, N), lambda b: (b, 0, 0)),
        pl.BlockSpec((1, Fn, N), lambda b: (b, 0, 0)),
        pl.BlockSpec((me, Fe), lambda b: (0, 0)),
        pl.BlockSpec((me, Fn), lambda b: (0, 0)),
        pl.BlockSpec((me, 1), lambda b: (0, 0)),
        pl.BlockSpec((me, 1), lambda b: (0, 0)),
    ]
    out_spec = pl.BlockSpec((1, Mout, N), lambda b: (b, 0, 0))

    flops = 2 * B * N * me * (Fe + Fn)
    bytes_accessed = B * N * 4 * (Fe + Fn + Mout)
    cost = pl.CostEstimate(flops=int(flops), transcendentals=0,
                           bytes_accessed=int(bytes_accessed))

    return pl.pallas_call(
        functools.partial(_msg_block_kernel, me=me),
        out_shape=out_shape,
        grid=grid,
        in_specs=in_specs,
        out_specs=out_spec,
        compiler_params=pltpu.CompilerParams(
            dimension_semantics=("parallel",)),
        cost_estimate=cost,
    )(e_vw, h_w, w_top, w_bot, b_top, b_bot)
